# baseline (speedup 1.0000x reference)
import functools
import numpy as np
import jax
import jax.numpy as jnp
from jax import lax
from jax.experimental import pallas as pl
from jax.experimental.pallas import tpu as pltpu
from jax.experimental.pallas import tpu_sc as plsc

_RATIO = 0.6
_ROT_FACTOR = 0.1


def _mask_params(B, H, W):
    rng = np.random.default_rng(0)
    lo = int(min(H * 0.5, W * 0.3))
    hi = int(max(H * 0.5, W * 0.3)) + 1
    ms = int(2 * max(H, W))
    rows = []
    for _ in range(B):
        g = int(rng.integers(lo, hi))
        if _RATIO == 1:
            l = int(rng.integers(1, g + 1))
        else:
            l = int(min(max(int(g * _RATIO + 0.5), 1), g - 1))
        s1 = int(rng.integers(0, g + 1))
        s2 = int(rng.integers(0, g + 1))
        ang = float(rng.uniform(-_ROT_FACTOR * 2.0 * np.pi,
                                _ROT_FACTOR * 2.0 * np.pi))
        n = ms // g
        rows.append([np.cos(ang), np.sin(ang), float(g), float(l),
                     float(n), float(s1), float(s2), 1.0 / g])
    return np.asarray(rows, dtype=np.float32)


def _floor_pos(v):
    return v.astype(jnp.int32).astype(jnp.float32)


def _stripe_pair(jf, sv, gv, lv, nv, rgv):
    t = jnp.maximum(jf - sv, 0.0)
    k0 = _floor_pos(t * rgv + 0.5)
    rem0 = t - k0 * gv
    wrapn = jnp.where(rem0 < 0.0, 1.0, 0.0)
    k = k0 - wrapn
    rem = rem0 + wrapn * gv
    ok0 = jnp.where((jf >= sv) & (rem < lv) & (k < nv), 1.0, 0.0)
    remp = rem + 1.0
    wrap = jnp.where(remp >= gv, 1.0, 0.0)
    rem1 = remp - wrap * gv
    k1 = k + wrap
    ok1 = jnp.where((jf + 1.0 >= sv) & (rem1 < lv) & (k1 < nv), 1.0, 0.0)
    return ok0, ok1


def _sc_kernel_fn(B, C, H, W, MS, RY, cy, offh, offw, NC,
                  x_hbm, p_hbm, o_hbm, pv, ibuf, obuf,
                  isem0, isem1, osem0, osem1):
    wid = lax.axis_index("s") * NC + lax.axis_index("c")
    nchunk = H // RY
    iota_f = lax.iota(jnp.int32, 16).astype(jnp.float32)
    isems = (isem0, isem1)
    osems = (osem0, osem1)

    @pl.when(wid < B)
    def _():
        b = wid
        pltpu.sync_copy(p_hbm.at[b], pv)
        ca = pv[0]
        sa = pv[1]
        gv = pv[2]
        lv = pv[3]
        nv = pv[4]
        s1 = pv[5]
        s2 = pv[6]
        rgv = pv[7]

        def start_in(kk, slot):
            ry = kk * RY
            for c in range(C):
                pltpu.async_copy(x_hbm.at[b, c, pl.ds(ry, RY)],
                                 ibuf.at[slot, c], isems[slot])

        def wait_in(kk, slot):
            ry = kk * RY
            for c in range(C):
                pltpu.make_async_copy(x_hbm.at[b, c, pl.ds(ry, RY)],
                                      ibuf.at[slot, c], isems[slot]).wait()

        def start_out(kk, slot):
            ry = kk * RY
            for c in range(C):
                pltpu.async_copy(obuf.at[slot, c],
                                 o_hbm.at[b, c, pl.ds(ry, RY)], osems[slot])

        def wait_out(kk, slot):
            ry = kk * RY
            for c in range(C):
                pltpu.make_async_copy(obuf.at[slot, c],
                                      o_hbm.at[b, c, pl.ds(ry, RY)],
                                      osems[slot]).wait()

        def compute(kk, slot):
            def row_body(rr, _):
                yf = (kk * RY + rr + offh).astype(jnp.float32) - cy
                ysb = ca * yf + cy
                xsb = -sa * yf + cy

                @plsc.parallel_loop(0, W // 16, unroll=4)
                def x_body(jx):
                    xf = iota_f + (jx * 16).astype(jnp.float32) + (offw - cy)
                    ys = sa * xf + ysb
                    xs = ca * xf + xsb
                    y0 = _floor_pos(ys)
                    x0 = _floor_pos(xs)
                    fy = ys - y0
                    fx = xs - x0
                    r0, r1 = _stripe_pair(y0, s1, gv, lv, nv, rgv)
                    c0, c1 = _stripe_pair(x0, s2, gv, lv, nv, rgv)
                    R = r0 + fy * (r1 - r0)
                    Cv = c0 + fx * (c1 - c0)
                    m = R + Cv - R * Cv
                    for c in range(C):
                        v = ibuf[slot, c, rr, pl.ds(jx * 16, 16)]
                        obuf[slot, c, rr, pl.ds(jx * 16, 16)] = v * m
                return 0

            lax.fori_loop(0, RY, row_body, 0)

        start_in(0, 0)
        if nchunk > 1:
            start_in(1, 1)

        def chunk2_body(k2, _):
            for par in range(2):
                kk = k2 * 2 + par
                wait_in(kk, par)

                @pl.when(kk >= 2)
                def _(kk=kk, par=par):
                    wait_out(kk - 2, par)

                compute(kk, par)
                start_out(kk, par)

                @pl.when(kk + 2 < nchunk)
                def _(kk=kk, par=par):
                    start_in(kk + 2, par)
            return 0

        lax.fori_loop(0, nchunk // 2, chunk2_body, 0)
        for par in range(2):
            if nchunk - 2 + par >= 0:
                wait_out(nchunk - 2 + par, par)


def kernel(images):
    B, H, W, C = images.shape
    ms = int(2 * max(H, W))
    cy = float((ms - 1) / 2.0)
    offh = (ms - H) // 2
    offw = (ms - W) // 2
    params = np.repeat(_mask_params(B, H, W)[:, :, None], 16, axis=2)
    params = jnp.asarray(params)

    RY = 8
    x = jnp.transpose(images, (0, 3, 1, 2))

    info = plsc.get_sparse_core_info()
    NC = info.num_cores
    mesh = plsc.VectorSubcoreMesh(core_axis_name="c", subcore_axis_name="s")
    body = functools.partial(_sc_kernel_fn, B, C, H, W, ms, RY, cy,
                             offh, offw, NC)
    sck = pl.kernel(
        body,
        mesh=mesh,
        out_type=jax.ShapeDtypeStruct((B, C, H, W), jnp.float32),
        scratch_types=[
            pltpu.VMEM((8, 16), jnp.float32),
            pltpu.VMEM((2, C, RY, W), jnp.float32),
            pltpu.VMEM((2, C, RY, W), jnp.float32),
            pltpu.SemaphoreType.DMA,
            pltpu.SemaphoreType.DMA,
            pltpu.SemaphoreType.DMA,
            pltpu.SemaphoreType.DMA,
        ],
    )
    out = sck(x, params)
    return jnp.transpose(out, (0, 2, 3, 1))


# speedup vs baseline: 1.6398x; 1.6398x over previous
import functools
import numpy as np
import jax
import jax.numpy as jnp
from jax import lax
from jax.experimental import pallas as pl
from jax.experimental.pallas import tpu as pltpu
from jax.experimental.pallas import tpu_sc as plsc

_RATIO = 0.6
_ROT_FACTOR = 0.1


def _mask_params(B, H, W):
    rng = np.random.default_rng(0)
    lo = int(min(H * 0.5, W * 0.3))
    hi = int(max(H * 0.5, W * 0.3)) + 1
    ms = int(2 * max(H, W))
    rows = []
    for _ in range(B):
        g = int(rng.integers(lo, hi))
        if _RATIO == 1:
            l = int(rng.integers(1, g + 1))
        else:
            l = int(min(max(int(g * _RATIO + 0.5), 1), g - 1))
        s1 = int(rng.integers(0, g + 1))
        s2 = int(rng.integers(0, g + 1))
        ang = float(rng.uniform(-_ROT_FACTOR * 2.0 * np.pi,
                                _ROT_FACTOR * 2.0 * np.pi))
        n = ms // g
        rows.append([np.cos(ang), np.sin(ang), float(g), float(l),
                     float(n), float(s1), float(s2), 1.0 / g])
    return np.asarray(rows, dtype=np.float32)


def _floor_pos(v):
    return v.astype(jnp.int32).astype(jnp.float32)


def _stripe_pair(jf, sv, gv, lv, nv, rgv):
    t = jnp.maximum(jf - sv, 0.0)
    k0 = _floor_pos(t * rgv + 0.5)
    rem0 = t - k0 * gv
    wrapn = jnp.where(rem0 < 0.0, 1.0, 0.0)
    k = k0 - wrapn
    rem = rem0 + wrapn * gv
    ok0 = jnp.where((jf >= sv) & (rem < lv) & (k < nv), 1.0, 0.0)
    remp = rem + 1.0
    wrap = jnp.where(remp >= gv, 1.0, 0.0)
    rem1 = remp - wrap * gv
    k1 = k + wrap
    ok1 = jnp.where((jf + 1.0 >= sv) & (rem1 < lv) & (k1 < nv), 1.0, 0.0)
    return ok0, ok1


def _sc_kernel_fn(B, C, H, W, MS, RY, cy, offh, offw, NC,
                  x_hbm, p_hbm, o_hbm, pv, ibuf, obuf,
                  isem0, isem1, osem0, osem1):
    wid = lax.axis_index("s") * NC + lax.axis_index("c")
    nchunk = H // RY
    iota_f = lax.iota(jnp.int32, 16).astype(jnp.float32)
    isems = (isem0, isem1)
    osems = (osem0, osem1)

    @pl.when(wid < B)
    def _():
        b = wid
        pltpu.sync_copy(p_hbm.at[b], pv)
        ca = pv[0]
        sa = pv[1]
        gv = pv[2]
        lv = pv[3]
        nv = pv[4]
        s1 = pv[5]
        s2 = pv[6]
        rgv = pv[7]

        def start_in(kk, slot):
            ry = kk * RY
            for c in range(C):
                pltpu.async_copy(x_hbm.at[b, c, pl.ds(ry, RY)],
                                 ibuf.at[slot, c], isems[slot])

        def wait_in(kk, slot):
            ry = kk * RY
            for c in range(C):
                pltpu.make_async_copy(x_hbm.at[b, c, pl.ds(ry, RY)],
                                      ibuf.at[slot, c], isems[slot]).wait()

        def start_out(kk, slot):
            ry = kk * RY
            for c in range(C):
                pltpu.async_copy(obuf.at[slot, c],
                                 o_hbm.at[b, c, pl.ds(ry, RY)], osems[slot])

        def wait_out(kk, slot):
            ry = kk * RY
            for c in range(C):
                pltpu.make_async_copy(obuf.at[slot, c],
                                      o_hbm.at[b, c, pl.ds(ry, RY)],
                                      osems[slot]).wait()

        def compute(kk, slot):
            def row_body(rr, _):
                yf = (kk * RY + rr + offh).astype(jnp.float32) - cy
                ysb = ca * yf + cy
                xsb = -sa * yf + cy

                @plsc.parallel_loop(0, W // 16, unroll=2)
                def x_body(jx):
                    xf = iota_f + (jx * 16).astype(jnp.float32) + (offw - cy)
                    ys = sa * xf + ysb
                    xs = ca * xf + xsb
                    y0 = _floor_pos(ys)
                    x0 = _floor_pos(xs)
                    fy = ys - y0
                    fx = xs - x0
                    r0, r1 = _stripe_pair(y0, s1, gv, lv, nv, rgv)
                    c0, c1 = _stripe_pair(x0, s2, gv, lv, nv, rgv)
                    R = r0 + fy * (r1 - r0)
                    Cv = c0 + fx * (c1 - c0)
                    m = R + Cv - R * Cv
                    for c in range(C):
                        v = ibuf[slot, c, rr, pl.ds(jx * 16, 16)]
                        obuf[slot, c, rr, pl.ds(jx * 16, 16)] = v * m
                return 0

            lax.fori_loop(0, RY, row_body, 0)

        start_in(0, 0)
        if nchunk > 1:
            start_in(1, 1)

        def chunk2_body(k2, _):
            for par in range(2):
                kk = k2 * 2 + par
                wait_in(kk, par)

                @pl.when(kk >= 2)
                def _(kk=kk, par=par):
                    wait_out(kk - 2, par)

                compute(kk, par)
                start_out(kk, par)

                @pl.when(kk + 2 < nchunk)
                def _(kk=kk, par=par):
                    start_in(kk + 2, par)
            return 0

        lax.fori_loop(0, nchunk // 2, chunk2_body, 0)
        for par in range(2):
            if nchunk - 2 + par >= 0:
                wait_out(nchunk - 2 + par, par)


def kernel(images):
    B, H, W, C = images.shape
    ms = int(2 * max(H, W))
    cy = float((ms - 1) / 2.0)
    offh = (ms - H) // 2
    offw = (ms - W) // 2
    params = np.repeat(_mask_params(B, H, W)[:, :, None], 16, axis=2)
    params = jnp.asarray(params)

    RY = 8
    x = jnp.transpose(images, (0, 3, 1, 2))

    info = plsc.get_sparse_core_info()
    NC = info.num_cores
    mesh = plsc.VectorSubcoreMesh(core_axis_name="c", subcore_axis_name="s")
    body = functools.partial(_sc_kernel_fn, B, C, H, W, ms, RY, cy,
                             offh, offw, NC)
    sck = pl.kernel(
        body,
        mesh=mesh,
        out_type=jax.ShapeDtypeStruct((B, C, H, W), jnp.float32),
        scratch_types=[
            pltpu.VMEM((8, 16), jnp.float32),
            pltpu.VMEM((2, C, RY, W), jnp.float32),
            pltpu.VMEM((2, C, RY, W), jnp.float32),
            pltpu.SemaphoreType.DMA,
            pltpu.SemaphoreType.DMA,
            pltpu.SemaphoreType.DMA,
            pltpu.SemaphoreType.DMA,
        ],
    )
    out = sck(x, params)
    return jnp.transpose(out, (0, 2, 3, 1))
